# Initial kernel scaffold; baseline (speedup 1.0000x reference)
import jax
import jax.numpy as jnp
from jax.experimental import pallas as pl


def kernel(scores_out, regression_out, gt_locations):
    raise NotImplementedError("write your pallas kernel here")


# speedup vs baseline: 3.0674x; 3.0674x over previous
import jax
import jax.numpy as jnp
from jax.experimental import pallas as pl
from jax.experimental.pallas import tpu as pltpu

H = 160
W = 128
N = H * W
KTOP = 2000
BUF = 2048
OUTN = 500
OUTP = 512
THR2 = 1.0
MIN_SCORE = 0.5
ONE_BITS = 0x3F800000

_f32 = jnp.float32


def _fiota(shape, dim):
    return jax.lax.broadcasted_iota(jnp.int32, shape, dim).astype(_f32)


def _dotT(a, b, exact=False):
    prec = jax.lax.Precision.HIGHEST if exact else jax.lax.Precision.DEFAULT
    return jax.lax.dot_general(
        a, b, (((1,), (1,)), ((), ())), precision=prec,
        preferred_element_type=_f32)


def _tril_strict(n):
    r = _fiota((n, n), 0)
    c = _fiota((n, n), 1)
    return (c < r).astype(_f32)


def _nms_body(scores_ref, reg_y_ref, reg_x_ref,
              out_s_ref, out_y_ref, out_x_ref,
              sel_s, pos_s, y_s, x_s, S_s):
    scores = scores_ref[0]
    ri = _fiota((H, W), 0)
    ci = _fiota((H, W), 1)
    y_s[...] = ri + 0.5 + reg_y_ref[0]
    x_s[...] = ci + 0.5 + reg_x_ref[0]

    s_bits = jax.lax.bitcast_convert_type(scores, jnp.int32)

    def bis(_, lohi):
        lo, hi = lohi
        m = lo + (hi - lo) // 2
        cnt = jnp.sum((s_bits >= m).astype(jnp.int32))
        big = cnt >= KTOP
        return (jnp.where(big, m, lo), jnp.where(big, hi, m))

    lo, _ = jax.lax.fori_loop(
        0, 31, bis, (jnp.int32(0), jnp.int32(ONE_BITS)))
    mask_gt = (s_bits > lo).astype(_f32)
    mask_eq = (s_bits == lo).astype(_f32)
    need_eq = _f32(KTOP) - jnp.sum(mask_gt)

    TT_W = _tril_strict(W)
    TL_H = _tril_strict(H)

    def excl2d(m2d):
        e = _dotT(m2d, TT_W)
        t = jnp.sum(m2d, axis=1, keepdims=True)
        roff = jax.lax.dot_general(
            TL_H, t, (((1,), (0,)), ((), ())), preferred_element_type=_f32)
        return roff + e

    eq_pos = excl2d(mask_eq)
    mask_sel = mask_gt + mask_eq * (eq_pos < need_eq).astype(_f32)
    sel_s[...] = mask_sel
    pos_s[...] = excl2d(mask_sel)

    iq_col = _fiota((BUF, 1), 0)
    lane_w = jax.lax.broadcasted_iota(jnp.int32, (1, W), 1)

    def crow(r, acc):
        s_r = scores_ref[0, pl.ds(r, 1), :]
        y_r = y_s[pl.ds(r, 1), :]
        x_r = x_s[pl.ds(r, 1), :]
        sel_r = sel_s[pl.ds(r, 1), :]
        pos_r = pos_s[pl.ds(r, 1), :]
        idx_r = (lane_w + r * W).astype(_f32)
        v_r = jnp.concatenate([s_r, y_r, x_r, idx_r], axis=0)
        o_r = (iq_col == pos_r).astype(_f32) * sel_r
        return acc + _dotT(v_r, o_r, exact=True)

    acc = jax.lax.fori_loop(0, H, crow, jnp.zeros((4, BUF), _f32))

    slot = _fiota((1, BUF), 1)
    real = slot < _f32(KTOP)
    sc = jnp.where(real, acc[0:1, :], -1.0)
    yc = jnp.where(real, acc[1:2, :], 1e6 + slot)
    xc = jnp.where(real, acc[2:3, :], 1e6)
    ic = jnp.where(real, acc[3:4, :], _f32(N) + slot)

    I_BUF = (_fiota((BUF, BUF), 0) ==
             _fiota((BUF, BUF), 1)).astype(_f32)

    def tocol(vrow):
        return _dotT(I_BUF, vrow, exact=True)

    s_col = tocol(sc)
    i_col = tocol(ic)
    g2 = (s_col > sc) | ((s_col == sc) & (i_col < ic))
    rank = jnp.sum(g2.astype(_f32), axis=0, keepdims=True)

    o_sort = (iq_col == rank).astype(_f32)
    vc = jnp.concatenate([sc, yc, xc], axis=0)
    vs = _dotT(vc, o_sort, exact=True)
    ss = vs[0:1, :]
    ys = vs[1:2, :]
    xs = vs[2:3, :]

    y_colS = tocol(ys)
    x_colS = tocol(xs)
    jrow = _fiota((1, BUF), 1)
    CH = 256
    for ib in range(BUF // CH):
        yb = jax.lax.slice(y_colS, (ib * CH, 0), ((ib + 1) * CH, 1))
        xb = jax.lax.slice(x_colS, (ib * CH, 0), ((ib + 1) * CH, 1))
        dy = yb - ys
        dx = xb - xs
        d2 = dy * dy + dx * dx
        i_col_b = (_fiota((CH, 1), 0)
                   + _f32(ib * CH))
        S_s[ib * CH:(ib + 1) * CH, :] = (
            (d2 < THR2) & (jrow > i_col_b)).astype(_f32)

    def nstep(i, keep):
        ki = jnp.sum(keep * (jrow == i.astype(_f32)).astype(_f32))
        row = S_s[pl.ds(i, 1), :]
        return keep * (1.0 - row * ki)

    keep = jax.lax.fori_loop(0, KTOP, nstep, jnp.ones((1, BUF), _f32))

    keepreal = keep * real.astype(_f32)
    kcnt = jnp.sum(keepreal)
    outpos = _dotT(keepreal, _tril_strict(BUF))
    oq_col = _fiota((OUTP, 1), 0)
    o_out = (oq_col == outpos).astype(_f32) * keepreal
    outv = _dotT(vs, o_out, exact=True)

    qi = _fiota((1, OUTP), 1)
    s_o = jnp.where(qi < kcnt, outv[0:1, :], -1.0)
    valid = s_o >= MIN_SCORE
    out_s_ref[0] = jnp.where(valid, s_o, -1.0)
    out_y_ref[0] = jnp.where(valid, outv[1:2, :], -1.0)
    out_x_ref[0] = jnp.where(valid, outv[2:3, :], -1.0)


def kernel(scores_out, regression_out, gt_locations):
    del gt_locations
    B = scores_out.shape[0]
    reg_y = regression_out[..., 0]
    reg_x = regression_out[..., 1]
    img_spec = pl.BlockSpec((1, H, W), lambda b: (b, 0, 0))
    out_spec = pl.BlockSpec((1, 1, OUTP), lambda b: (b, 0, 0))
    s, y, x = pl.pallas_call(
        _nms_body,
        grid=(B,),
        in_specs=[img_spec, img_spec, img_spec],
        out_specs=[out_spec, out_spec, out_spec],
        out_shape=[jax.ShapeDtypeStruct((B, 1, OUTP), _f32)] * 3,
        scratch_shapes=[pltpu.VMEM((H, W), _f32)] * 4 +
                       [pltpu.VMEM((BUF, BUF), _f32)],
        compiler_params=pltpu.CompilerParams(
            vmem_limit_bytes=128 * 1024 * 1024),
    )(scores_out, reg_y, reg_x)
    locs = jnp.stack([y[:, 0, :OUTN], x[:, 0, :OUTN]], axis=-1)
    return locs, s[:, 0, :OUTN]


# speedup vs baseline: 7.6822x; 2.5044x over previous
import jax
import jax.numpy as jnp
from jax.experimental import pallas as pl
from jax.experimental.pallas import tpu as pltpu

H = 160
W = 128
N = H * W
KTOP = 2000
BUF = 2048
OUTN = 500
OUTP = 512
THR2 = 1.0
MIN_SCORE = 0.5
ONE_BITS = 0x3F800000

_f32 = jnp.float32


def _fiota(shape, dim):
    return jax.lax.broadcasted_iota(jnp.int32, shape, dim).astype(_f32)


def _dotT(a, b, exact=False):
    prec = jax.lax.Precision.HIGHEST if exact else jax.lax.Precision.DEFAULT
    return jax.lax.dot_general(
        a, b, (((1,), (1,)), ((), ())), precision=prec,
        preferred_element_type=_f32)


def _tril_strict(n):
    r = _fiota((n, n), 0)
    c = _fiota((n, n), 1)
    return (c < r).astype(_f32)


def _nms_body(scores_ref, reg_y_ref, reg_x_ref,
              out_s_ref, out_y_ref, out_x_ref,
              sel_s, pos_s, y_s, x_s, S_s, keep_s):
    scores = scores_ref[0]
    ri = _fiota((H, W), 0)
    ci = _fiota((H, W), 1)
    y_s[...] = ri + 0.5 + reg_y_ref[0]
    x_s[...] = ci + 0.5 + reg_x_ref[0]

    s_bits = jax.lax.bitcast_convert_type(scores, jnp.int32)

    def bis(_, lohi):
        lo, hi = lohi
        m = lo + (hi - lo) // 2
        cnt = jnp.sum((s_bits >= m).astype(jnp.int32))
        big = cnt >= KTOP
        return (jnp.where(big, m, lo), jnp.where(big, hi, m))

    lo, _ = jax.lax.fori_loop(
        0, 31, bis, (jnp.int32(0), jnp.int32(ONE_BITS)))
    mask_gt = (s_bits > lo).astype(_f32)
    mask_eq = (s_bits == lo).astype(_f32)
    need_eq = _f32(KTOP) - jnp.sum(mask_gt)

    TT_W = _tril_strict(W)
    TL_H = _tril_strict(H)

    def excl2d(m2d):
        e = _dotT(m2d, TT_W)
        t = jnp.sum(m2d, axis=1, keepdims=True)
        roff = jax.lax.dot_general(
            TL_H, t, (((1,), (0,)), ((), ())), preferred_element_type=_f32)
        return roff + e

    eq_pos = excl2d(mask_eq)
    mask_sel = mask_gt + mask_eq * (eq_pos < need_eq).astype(_f32)
    sel_s[...] = mask_sel
    pos_s[...] = excl2d(mask_sel)

    iq_col = _fiota((BUF, 1), 0)
    lane_w = jax.lax.broadcasted_iota(jnp.int32, (1, W), 1)

    def crow(r, acc):
        s_r = scores_ref[0, pl.ds(r, 1), :]
        y_r = y_s[pl.ds(r, 1), :]
        x_r = x_s[pl.ds(r, 1), :]
        sel_r = sel_s[pl.ds(r, 1), :]
        pos_r = pos_s[pl.ds(r, 1), :]
        idx_r = (lane_w + r * W).astype(_f32)
        v_r = jnp.concatenate([s_r, y_r, x_r, idx_r], axis=0)
        o_r = (iq_col == pos_r).astype(_f32) * sel_r
        return acc + _dotT(v_r, o_r, exact=True)

    acc = jax.lax.fori_loop(0, H, crow, jnp.zeros((4, BUF), _f32))

    slot = _fiota((1, BUF), 1)
    real = slot < _f32(KTOP)
    sc = jnp.where(real, acc[0:1, :], -1.0)
    yc = jnp.where(real, acc[1:2, :], 1e6 + slot)
    xc = jnp.where(real, acc[2:3, :], 1e6)
    ic = jnp.where(real, acc[3:4, :], _f32(N) + slot)

    I_BUF = (_fiota((BUF, BUF), 0) ==
             _fiota((BUF, BUF), 1)).astype(_f32)

    def tocol(vrow):
        return _dotT(I_BUF, vrow, exact=True)

    s_col = tocol(sc)
    i_col = tocol(ic)
    g2 = (s_col > sc) | ((s_col == sc) & (i_col < ic))
    rank = jnp.sum(g2.astype(_f32), axis=0, keepdims=True)

    o_sort = (iq_col == rank).astype(_f32)
    vc = jnp.concatenate([sc, yc, xc], axis=0)
    vs = _dotT(vc, o_sort, exact=True)
    ss = vs[0:1, :]
    ys = vs[1:2, :]
    xs = vs[2:3, :]

    y_colS = tocol(ys)
    x_colS = tocol(xs)
    jrow = _fiota((1, BUF), 1)
    CH = 256
    for ib in range(BUF // CH):
        yb = jax.lax.slice(y_colS, (ib * CH, 0), ((ib + 1) * CH, 1))
        xb = jax.lax.slice(x_colS, (ib * CH, 0), ((ib + 1) * CH, 1))
        dy = yb - ys
        dx = xb - xs
        d2 = dy * dy + dx * dx
        i_col_b = (_fiota((CH, 1), 0)
                   + _f32(ib * CH))
        S_s[ib * CH:(ib + 1) * CH, :] = (
            (d2 < THR2) & (jrow > i_col_b)).astype(_f32)

    keep_s[...] = jnp.ones((1, BUF), _f32)

    def ncond(st):
        changed, t = st
        return jnp.logical_and(changed, t < BUF + 4)

    def nbody(st):
        _, t = st
        k_cur = keep_s[...]
        sup = jax.lax.dot_general(
            k_cur, S_s[...], (((1,), (0,)), ((), ())),
            preferred_element_type=_f32)
        k_new = (sup < 0.5).astype(_f32)
        keep_s[...] = k_new
        return (jnp.sum(jnp.abs(k_new - k_cur)) > 0.0, t + 1)

    jax.lax.while_loop(ncond, nbody, (True, jnp.int32(0)))
    keep = keep_s[...]

    keepreal = keep * real.astype(_f32)
    kcnt = jnp.sum(keepreal)
    outpos = _dotT(keepreal, _tril_strict(BUF))
    oq_col = _fiota((OUTP, 1), 0)
    o_out = (oq_col == outpos).astype(_f32) * keepreal
    outv = _dotT(vs, o_out, exact=True)

    qi = _fiota((1, OUTP), 1)
    s_o = jnp.where(qi < kcnt, outv[0:1, :], -1.0)
    valid = s_o >= MIN_SCORE
    out_s_ref[0] = jnp.where(valid, s_o, -1.0)
    out_y_ref[0] = jnp.where(valid, outv[1:2, :], -1.0)
    out_x_ref[0] = jnp.where(valid, outv[2:3, :], -1.0)


def kernel(scores_out, regression_out, gt_locations):
    del gt_locations
    B = scores_out.shape[0]
    reg_y = regression_out[..., 0]
    reg_x = regression_out[..., 1]
    img_spec = pl.BlockSpec((1, H, W), lambda b: (b, 0, 0))
    out_spec = pl.BlockSpec((1, 1, OUTP), lambda b: (b, 0, 0))
    s, y, x = pl.pallas_call(
        _nms_body,
        grid=(B,),
        in_specs=[img_spec, img_spec, img_spec],
        out_specs=[out_spec, out_spec, out_spec],
        out_shape=[jax.ShapeDtypeStruct((B, 1, OUTP), _f32)] * 3,
        scratch_shapes=[pltpu.VMEM((H, W), _f32)] * 4 +
                       [pltpu.VMEM((BUF, BUF), _f32),
                        pltpu.VMEM((1, BUF), _f32)],
        compiler_params=pltpu.CompilerParams(
            vmem_limit_bytes=128 * 1024 * 1024),
    )(scores_out, reg_y, reg_x)
    locs = jnp.stack([y[:, 0, :OUTN], x[:, 0, :OUTN]], axis=-1)
    return locs, s[:, 0, :OUTN]


# speedup vs baseline: 18.5747x; 2.4179x over previous
import jax
import jax.numpy as jnp
from jax.experimental import pallas as pl
from jax.experimental.pallas import tpu as pltpu

H = 160
W = 128
N = H * W
KTOP = 2000
BUF = 2048
OUTN = 500
OUTP = 512
THR2 = 1.0
MIN_SCORE = 0.5
ONE_BITS = 0x3F800000

_f32 = jnp.float32


def _fiota(shape, dim):
    return jax.lax.broadcasted_iota(jnp.int32, shape, dim).astype(_f32)


def _dotT(a, b, exact=False):
    prec = jax.lax.Precision.HIGHEST if exact else jax.lax.Precision.DEFAULT
    return jax.lax.dot_general(
        a, b, (((1,), (1,)), ((), ())), precision=prec,
        preferred_element_type=_f32)


def _tril_strict(n):
    r = _fiota((n, n), 0)
    c = _fiota((n, n), 1)
    return (c < r).astype(_f32)


def _nms_body(scores_ref, reg_y_ref, reg_x_ref,
              out_s_ref, out_y_ref, out_x_ref,
              sel_s, pos_s, y_s, x_s, S_s, keep_s, acc_s):
    scores = scores_ref[0]
    ri = _fiota((H, W), 0)
    ci = _fiota((H, W), 1)
    y_s[...] = ri + 0.5 + reg_y_ref[0]
    x_s[...] = ci + 0.5 + reg_x_ref[0]

    s_bits = jax.lax.bitcast_convert_type(scores, jnp.int32)

    def bis(_, lohi):
        lo, hi = lohi
        m = lo + (hi - lo) // 2
        cnt = jnp.sum((s_bits >= m).astype(jnp.int32))
        big = cnt >= KTOP
        return (jnp.where(big, m, lo), jnp.where(big, hi, m))

    lo, _ = jax.lax.fori_loop(
        0, 31, bis, (jnp.int32(0), jnp.int32(ONE_BITS)))
    mask_gt = (s_bits > lo).astype(_f32)
    mask_eq = (s_bits == lo).astype(_f32)
    need_eq = _f32(KTOP) - jnp.sum(mask_gt)

    TT_W = _tril_strict(W)
    TL_H = _tril_strict(H)

    def excl2d(m2d):
        e = _dotT(m2d, TT_W)
        t = jnp.sum(m2d, axis=1, keepdims=True)
        roff = jax.lax.dot_general(
            TL_H, t, (((1,), (0,)), ((), ())), preferred_element_type=_f32)
        return roff + e

    eq_pos = excl2d(mask_eq)
    mask_sel = mask_gt + mask_eq * (eq_pos < need_eq).astype(_f32)
    sel_s[...] = mask_sel
    pos_s[...] = _dotT(mask_sel, TT_W)

    iq256 = _fiota((2 * W, 1), 0)
    acc_s[...] = jnp.zeros((3, BUF // W + 1, W), _f32)

    def crow(r, roff):
        s_r = scores_ref[0, pl.ds(r, 1), :]
        y_r = y_s[pl.ds(r, 1), :]
        x_r = x_s[pl.ds(r, 1), :]
        sel_r = sel_s[pl.ds(r, 1), :]
        e_r = pos_s[pl.ds(r, 1), :]
        hi = (roff.astype(jnp.int32)) // W
        lo = roff - (hi * W).astype(_f32)
        v_r = jnp.concatenate([s_r, y_r, x_r], axis=0)
        o_r = (iq256 == (e_r + lo)).astype(_f32) * sel_r
        part = _dotT(v_r, o_r, exact=True)
        p0 = part[:, 0:W].reshape(3, 1, W)
        p1 = part[:, W:2 * W].reshape(3, 1, W)
        acc_s[:, pl.ds(hi, 1), :] += p0
        acc_s[:, pl.ds(hi + 1, 1), :] += p1
        return roff + jnp.sum(sel_r)

    jax.lax.fori_loop(0, H, crow, _f32(0.0))
    accv = acc_s[...]
    flat = jnp.concatenate(
        [accv[:, g, :] for g in range(BUF // W)], axis=1)

    iq_col = _fiota((BUF, 1), 0)
    slot = _fiota((1, BUF), 1)
    real = slot < _f32(KTOP)
    sc = jnp.where(real, flat[0:1, :], -1.0)
    yc = jnp.where(real, flat[1:2, :], 1e6 + slot)
    xc = jnp.where(real, flat[2:3, :], 1e6)

    I_BUF = (_fiota((BUF, BUF), 0) ==
             _fiota((BUF, BUF), 1)).astype(_f32)

    def tocol(vrow):
        return _dotT(I_BUF, vrow, exact=True)

    s_col = tocol(sc)
    g2 = (s_col > sc) | ((s_col == sc) & (iq_col < slot))
    rank = jnp.sum(g2.astype(_f32), axis=0, keepdims=True)

    o_sort = (iq_col == rank).astype(_f32)
    vc = jnp.concatenate([sc, yc, xc], axis=0)
    vs = _dotT(vc, o_sort, exact=True)
    ss = vs[0:1, :]
    ys = vs[1:2, :]
    xs = vs[2:3, :]

    y_colS = tocol(ys)
    x_colS = tocol(xs)
    jrow = _fiota((1, BUF), 1)
    CH = 256
    for ib in range(BUF // CH):
        yb = jax.lax.slice(y_colS, (ib * CH, 0), ((ib + 1) * CH, 1))
        xb = jax.lax.slice(x_colS, (ib * CH, 0), ((ib + 1) * CH, 1))
        dy = yb - ys
        dx = xb - xs
        d2 = dy * dy + dx * dx
        i_col_b = (_fiota((CH, 1), 0)
                   + _f32(ib * CH))
        S_s[ib * CH:(ib + 1) * CH, :] = (
            (d2 < THR2) & (jrow > i_col_b)).astype(_f32)

    keep_s[...] = jnp.ones((1, BUF), _f32)

    def ncond(st):
        changed, t = st
        return jnp.logical_and(changed, t < BUF + 4)

    def nbody(st):
        _, t = st
        k_cur = keep_s[...]
        sup = jax.lax.dot_general(
            k_cur, S_s[...], (((1,), (0,)), ((), ())),
            preferred_element_type=_f32)
        k_new = (sup < 0.5).astype(_f32)
        keep_s[...] = k_new
        return (jnp.sum(jnp.abs(k_new - k_cur)) > 0.0, t + 1)

    jax.lax.while_loop(ncond, nbody, (True, jnp.int32(0)))
    keep = keep_s[...]

    keepreal = keep * real.astype(_f32)
    kcnt = jnp.sum(keepreal)
    outpos = _dotT(keepreal, _tril_strict(BUF))
    oq_col = _fiota((OUTP, 1), 0)
    o_out = (oq_col == outpos).astype(_f32) * keepreal
    outv = _dotT(vs, o_out, exact=True)

    qi = _fiota((1, OUTP), 1)
    s_o = jnp.where(qi < kcnt, outv[0:1, :], -1.0)
    valid = s_o >= MIN_SCORE
    out_s_ref[0] = jnp.where(valid, s_o, -1.0)
    out_y_ref[0] = jnp.where(valid, outv[1:2, :], -1.0)
    out_x_ref[0] = jnp.where(valid, outv[2:3, :], -1.0)


def kernel(scores_out, regression_out, gt_locations):
    del gt_locations
    B = scores_out.shape[0]
    reg_y = regression_out[..., 0]
    reg_x = regression_out[..., 1]
    img_spec = pl.BlockSpec((1, H, W), lambda b: (b, 0, 0))
    out_spec = pl.BlockSpec((1, 1, OUTP), lambda b: (b, 0, 0))
    s, y, x = pl.pallas_call(
        _nms_body,
        grid=(B,),
        in_specs=[img_spec, img_spec, img_spec],
        out_specs=[out_spec, out_spec, out_spec],
        out_shape=[jax.ShapeDtypeStruct((B, 1, OUTP), _f32)] * 3,
        scratch_shapes=[pltpu.VMEM((H, W), _f32)] * 4 +
                       [pltpu.VMEM((BUF, BUF), _f32),
                        pltpu.VMEM((1, BUF), _f32),
                        pltpu.VMEM((3, BUF // W + 1, W), _f32)],
        compiler_params=pltpu.CompilerParams(
            vmem_limit_bytes=128 * 1024 * 1024),
    )(scores_out, reg_y, reg_x)
    locs = jnp.stack([y[:, 0, :OUTN], x[:, 0, :OUTN]], axis=-1)
    return locs, s[:, 0, :OUTN]


# speedup vs baseline: 19.7386x; 1.0627x over previous
import jax
import jax.numpy as jnp
from jax.experimental import pallas as pl
from jax.experimental.pallas import tpu as pltpu

H = 160
W = 128
N = H * W
KTOP = 2000
BUF = 2048
OUTN = 500
OUTP = 512
THR2 = 1.0
MIN_SCORE = 0.5
ONE_BITS = 0x3F800000

_f32 = jnp.float32


def _fiota(shape, dim):
    return jax.lax.broadcasted_iota(jnp.int32, shape, dim).astype(_f32)


def _dotT(a, b, exact=False):
    prec = jax.lax.Precision.HIGHEST if exact else jax.lax.Precision.DEFAULT
    return jax.lax.dot_general(
        a, b, (((1,), (1,)), ((), ())), precision=prec,
        preferred_element_type=_f32)


def _tril_strict(n):
    r = _fiota((n, n), 0)
    c = _fiota((n, n), 1)
    return (c < r).astype(_f32)


def _nms_body(scores_ref, reg_y_ref, reg_x_ref,
              out_s_ref, out_y_ref, out_x_ref,
              sel_s, pos_s, y_s, x_s, S_s, keep_s, acc_s):
    scores = scores_ref[0]
    ri = _fiota((H, W), 0)
    ci = _fiota((H, W), 1)
    y_s[...] = ri + 0.5 + reg_y_ref[0]
    x_s[...] = ci + 0.5 + reg_x_ref[0]

    s_bits = jax.lax.bitcast_convert_type(scores, jnp.int32)

    def bis(_, lohi):
        lo, hi = lohi
        m = lo + (hi - lo) // 2
        cnt = jnp.sum((s_bits >= m).astype(jnp.int32))
        big = cnt >= KTOP
        return (jnp.where(big, m, lo), jnp.where(big, hi, m))

    lo, _ = jax.lax.fori_loop(
        0, 31, bis, (jnp.int32(0), jnp.int32(ONE_BITS)))
    mask_gt = (s_bits > lo).astype(_f32)
    mask_eq = (s_bits == lo).astype(_f32)
    need_eq = _f32(KTOP) - jnp.sum(mask_gt)

    TT_W = _tril_strict(W)
    TL_H = _tril_strict(H)

    def excl2d(m2d):
        e = _dotT(m2d, TT_W)
        t = jnp.sum(m2d, axis=1, keepdims=True)
        roff = jax.lax.dot_general(
            TL_H, t, (((1,), (0,)), ((), ())), preferred_element_type=_f32)
        return roff + e

    eq_pos = excl2d(mask_eq)
    mask_sel = mask_gt + mask_eq * (eq_pos < need_eq).astype(_f32)
    sel_s[...] = mask_sel
    pos_s[...] = _dotT(mask_sel, TT_W)

    iq256 = _fiota((2 * W, 1), 0)
    acc_s[...] = jnp.zeros((3, BUF // W + 1, W), _f32)

    def crow(r, roff):
        s_r = scores_ref[0, pl.ds(r, 1), :]
        y_r = y_s[pl.ds(r, 1), :]
        x_r = x_s[pl.ds(r, 1), :]
        sel_r = sel_s[pl.ds(r, 1), :]
        e_r = pos_s[pl.ds(r, 1), :]
        hi = (roff.astype(jnp.int32)) // W
        lo = roff - (hi * W).astype(_f32)
        v_r = jnp.concatenate([s_r, y_r, x_r], axis=0)
        o_r = (iq256 == (e_r + lo)).astype(_f32) * sel_r
        part = _dotT(v_r, o_r, exact=True)
        p0 = part[:, 0:W].reshape(3, 1, W)
        p1 = part[:, W:2 * W].reshape(3, 1, W)
        acc_s[:, pl.ds(hi, 1), :] += p0
        acc_s[:, pl.ds(hi + 1, 1), :] += p1
        return roff + jnp.sum(sel_r)

    jax.lax.fori_loop(0, H, crow, _f32(0.0))
    accv = acc_s[...]
    flat = jnp.concatenate(
        [accv[:, g, :] for g in range(BUF // W)], axis=1)

    iq_col = _fiota((BUF, 1), 0)
    slot = _fiota((1, BUF), 1)
    real = slot < _f32(KTOP)
    sc = jnp.where(real, flat[0:1, :], -1.0)
    yc = jnp.where(real, flat[1:2, :], 1e6 + slot)
    xc = jnp.where(real, flat[2:3, :], 1e6)

    I_BUF = (_fiota((BUF, BUF), 0) ==
             _fiota((BUF, BUF), 1)).astype(_f32)
    vc = jnp.concatenate([sc, yc, xc], axis=0)
    cols = _dotT(I_BUF, vc, exact=True)
    s_col = cols[:, 0:1]
    y_col = cols[:, 1:2]
    x_col = cols[:, 2:3]
    g2 = (s_col > sc) | ((s_col == sc) & (iq_col < slot))
    g2f = g2.astype(_f32)

    CH = 256
    for ib in range(BUF // CH):
        yb = jax.lax.slice(y_col, (ib * CH, 0), ((ib + 1) * CH, 1))
        xb = jax.lax.slice(x_col, (ib * CH, 0), ((ib + 1) * CH, 1))
        dy = yb - yc
        dx = xb - xc
        d2 = dy * dy + dx * dx
        g2b = jax.lax.slice(g2, (ib * CH, 0), ((ib + 1) * CH, BUF))
        S_s[ib * CH:(ib + 1) * CH, :] = ((d2 < THR2) & g2b).astype(_f32)

    keep_s[...] = jnp.ones((1, BUF), _f32)

    def ncond(st):
        changed, t = st
        return jnp.logical_and(changed, t < BUF + 4)

    def nbody(st):
        _, t = st
        k_cur = keep_s[...]
        sup = jax.lax.dot_general(
            k_cur, S_s[...], (((1,), (0,)), ((), ())),
            preferred_element_type=_f32)
        k_new = (sup < 0.5).astype(_f32)
        keep_s[...] = k_new
        return (jnp.sum(jnp.abs(k_new - k_cur)) > 0.0, t + 1)

    jax.lax.while_loop(ncond, nbody, (True, jnp.int32(0)))
    keep = keep_s[...]

    keepreal = keep * real.astype(_f32)
    kcnt = jnp.sum(keepreal)
    outpos = jax.lax.dot_general(
        keepreal, g2f, (((1,), (0,)), ((), ())),
        preferred_element_type=_f32)
    oq_col = _fiota((OUTP, 1), 0)
    o_out = (oq_col == outpos).astype(_f32) * keepreal
    outv = _dotT(vc, o_out, exact=True)

    qi = _fiota((1, OUTP), 1)
    s_o = jnp.where(qi < kcnt, outv[0:1, :], -1.0)
    valid = s_o >= MIN_SCORE
    out_s_ref[0] = jnp.where(valid, s_o, -1.0)
    out_y_ref[0] = jnp.where(valid, outv[1:2, :], -1.0)
    out_x_ref[0] = jnp.where(valid, outv[2:3, :], -1.0)


def kernel(scores_out, regression_out, gt_locations):
    del gt_locations
    B = scores_out.shape[0]
    reg_y = regression_out[..., 0]
    reg_x = regression_out[..., 1]
    img_spec = pl.BlockSpec((1, H, W), lambda b: (b, 0, 0))
    out_spec = pl.BlockSpec((1, 1, OUTP), lambda b: (b, 0, 0))
    s, y, x = pl.pallas_call(
        _nms_body,
        grid=(B,),
        in_specs=[img_spec, img_spec, img_spec],
        out_specs=[out_spec, out_spec, out_spec],
        out_shape=[jax.ShapeDtypeStruct((B, 1, OUTP), _f32)] * 3,
        scratch_shapes=[pltpu.VMEM((H, W), _f32)] * 4 +
                       [pltpu.VMEM((BUF, BUF), _f32),
                        pltpu.VMEM((1, BUF), _f32),
                        pltpu.VMEM((3, BUF // W + 1, W), _f32)],
        compiler_params=pltpu.CompilerParams(
            vmem_limit_bytes=128 * 1024 * 1024),
    )(scores_out, reg_y, reg_x)
    locs = jnp.stack([y[:, 0, :OUTN], x[:, 0, :OUTN]], axis=-1)
    return locs, s[:, 0, :OUTN]
